# Initial kernel scaffold; baseline (speedup 1.0000x reference)
import jax
import jax.numpy as jnp
from jax.experimental import pallas as pl


def kernel(x, segment_tokens, token_table, segment_table, pe):
    raise NotImplementedError("write your pallas kernel here")


# speedup vs baseline: 5.6529x; 5.6529x over previous
import functools

import jax
import jax.numpy as jnp
from jax import lax
from jax.experimental import pallas as pl
from jax.experimental.pallas import tpu as pltpu
from jax.experimental.pallas import tpu_sc as plsc

B, S, D, V = 1024, 200, 128, 100000
N = B * S
NC, NS = 2, 16
NW = NC * NS
RPW = N // NW
C = 128
NCHUNK = RPW // C


def _combine_body(pe_ref, seg_ref, out_ref):
    out_ref[...] = seg_ref[...][:, None, :] + pe_ref[...][None, :, :]


def _make_combined(pe, segment_table):
    out = pl.pallas_call(
        _combine_body,
        out_shape=jax.ShapeDtypeStruct((2, S, D), jnp.float32),
    )(pe, segment_table)
    return out.reshape(2 * S, D)


def _sc_body(x_hbm, seg_hbm, tok_tab_hbm, comb_hbm, out_hbm,
             xidx_v, seg_v, idx2_v, tok_v, add_v, sem1, sem2):
    wid = lax.axis_index("s") * NC + lax.axis_index("c")
    base = wid * RPW

    def chunk_body(c, carry):
        gbase = base + c * C
        pltpu.sync_copy(x_hbm.at[pl.ds(gbase, C)], xidx_v)
        cp1 = pltpu.async_copy(tok_tab_hbm.at[xidx_v], tok_v, sem1)
        pltpu.sync_copy(seg_hbm.at[pl.ds(gbase, C)], seg_v)
        for j in range(C // 16):
            rowid = gbase + j * 16 + lax.iota(jnp.int32, 16)
            pos = lax.rem(rowid, S)
            idx2_v[pl.ds(j * 16, 16)] = seg_v[pl.ds(j * 16, 16)] * S + pos
        cp2 = pltpu.async_copy(comb_hbm.at[idx2_v], add_v, sem2)
        cp1.wait()
        cp2.wait()

        def row_body(i, acc):
            for u in range(4):
                r = i * 4 + u
                for j in range(D // 16):
                    sl = pl.ds(j * 16, 16)
                    plsc.addupdate(tok_v.at[r, sl], add_v[r, sl])
            return acc

        lax.fori_loop(0, C // 4, row_body, 0)
        pltpu.sync_copy(tok_v, out_hbm.at[pl.ds(gbase, C)])
        return carry

    lax.fori_loop(0, NCHUNK, chunk_body, 0)


def kernel(x, segment_tokens, token_table, segment_table, pe):
    combined = _make_combined(pe.astype(jnp.float32),
                              segment_table.astype(jnp.float32))
    x_flat = x.reshape(N).astype(jnp.int32)
    seg_flat = segment_tokens.reshape(N).astype(jnp.int32)

    mesh = plsc.VectorSubcoreMesh(core_axis_name="c", subcore_axis_name="s")
    sc = functools.partial(
        pl.kernel,
        mesh=mesh,
        out_type=jax.ShapeDtypeStruct((N, D), jnp.float32),
        scratch_types=[
            pltpu.VMEM((C,), jnp.int32),
            pltpu.VMEM((C,), jnp.int32),
            pltpu.VMEM((C,), jnp.int32),
            pltpu.VMEM((C, D), jnp.float32),
            pltpu.VMEM((C, D), jnp.float32),
            pltpu.SemaphoreType.DMA,
            pltpu.SemaphoreType.DMA,
        ],
    )(_sc_body)
    out = sc(x_flat, seg_flat, token_table, combined)
    return out.reshape(B, S, D)
